# Initial kernel scaffold; baseline (speedup 1.0000x reference)
import jax
import jax.numpy as jnp
from jax.experimental import pallas as pl


def kernel(x, edge_index, params):
    raise NotImplementedError("write your pallas kernel here")


# speedup vs baseline: 2.7635x; 2.7635x over previous
import functools

import jax
import jax.numpy as jnp
from jax import lax
from jax.experimental import pallas as pl
from jax.experimental.pallas import tpu as pltpu
from jax.experimental.pallas import tpu_sc as plsc

N = 10000
E = 160000
NP = 10240
LANES = 128
NC = 2
NS = 16
L = 16
EB = 80
RPT = NP // NS
ZR = 128
BN = 1000
GB = N // BN
F32 = jnp.float32


def _sc_body_factory(n_chunks):
    cpc = n_chunks // NC
    et = E // NS
    nb = et // EB

    def body(h_hbm, srcoff_hbm, dst_hbm, out_hbm,
             acc, idx_s, idx_d, rows, zb, sem):
        cid = lax.axis_index("c")
        sid = lax.axis_index("s")
        row0 = pl.multiple_of(sid * RPT, ZR)
        zeros16 = jnp.zeros((L,), F32)

        def _zrow(r, _):
            def _zlane(q, _):
                zb[r, pl.ds(q * L, L)] = zeros16
                return 0
            return lax.fori_loop(0, LANES // L, _zlane, 0)
        lax.fori_loop(0, ZR, _zrow, 0)

        for j in range(cpc):
            chunk = cid + NC * j

            for k in range(RPT // ZR):
                pltpu.sync_copy(zb, acc.at[pl.ds(row0 + k * ZR, ZR)])
            plsc.subcore_barrier()

            def _edge(i, _):
                e0 = chunk * E + sid * et + i * EB
                d0 = sid * et + i * EB
                pltpu.sync_copy(srcoff_hbm.at[pl.ds(e0, EB)], idx_s)
                pltpu.sync_copy(dst_hbm.at[pl.ds(d0, EB)], idx_d)
                pltpu.async_copy(h_hbm.at[idx_s], rows, sem).wait()
                pltpu.sync_copy(rows, acc.at[idx_d], add=True)
                return 0
            lax.fori_loop(0, nb, _edge, 0)
            plsc.subcore_barrier()

            base = pl.multiple_of(chunk * NP + row0, ZR)
            for k in range(RPT // ZR):
                pltpu.sync_copy(
                    acc.at[pl.ds(row0 + k * ZR, ZR)],
                    out_hbm.at[pl.ds(base + k * ZR, ZR)])

    return body


@functools.lru_cache(maxsize=None)
def _make_sc_agg(n_chunks):
    mesh = plsc.VectorSubcoreMesh(core_axis_name="c", subcore_axis_name="s")
    return pl.kernel(
        _sc_body_factory(n_chunks),
        out_type=jax.ShapeDtypeStruct((n_chunks * NP, LANES), F32),
        mesh=mesh,
        scratch_types=(
            pltpu.VMEM_SHARED((NP, LANES), F32),
            pltpu.VMEM((EB,), jnp.int32),
            pltpu.VMEM((EB,), jnp.int32),
            pltpu.VMEM((EB, LANES), F32),
            pltpu.VMEM((ZR, LANES), F32),
            pltpu.SemaphoreType.DMA,
        ),
    )


EBD = 40


def _sc_deg_body(dst_hbm, deg_hbm, acc, idx_d, ones, zb):
    cid = lax.axis_index("c")
    sid = lax.axis_index("s")
    row0 = pl.multiple_of(sid * RPT, ZR)
    et = E // (NC * NS)
    nb = et // EBD
    zeros16 = jnp.zeros((L,), F32)
    ones16 = jnp.ones((L,), F32)

    def _fill(ref, nrows, val):
        def _row(r, _):
            def _lane(q, _):
                ref[r, pl.ds(q * L, L)] = val
                return 0
            return lax.fori_loop(0, LANES // L, _lane, 0)
        lax.fori_loop(0, nrows, _row, 0)

    _fill(ones, EBD, ones16)
    _fill(zb, ZR, zeros16)
    for k in range(RPT // ZR):
        pltpu.sync_copy(zb, acc.at[pl.ds(row0 + k * ZR, ZR)])
    plsc.subcore_barrier()

    def _edge(i, _):
        d0 = cid * (E // NC) + sid * et + i * EBD
        pltpu.sync_copy(dst_hbm.at[pl.ds(d0, EBD)], idx_d)
        pltpu.sync_copy(ones, acc.at[idx_d], add=True)
        return 0
    lax.fori_loop(0, nb, _edge, 0)
    plsc.subcore_barrier()

    base = pl.multiple_of(cid * NP + row0, ZR)
    for k in range(RPT // ZR):
        pltpu.sync_copy(acc.at[pl.ds(row0 + k * ZR, ZR)],
                        deg_hbm.at[pl.ds(base + k * ZR, ZR)])


def _make_sc_deg():
    mesh = plsc.VectorSubcoreMesh(core_axis_name="c", subcore_axis_name="s")
    return pl.kernel(
        _sc_deg_body,
        out_type=jax.ShapeDtypeStruct((NC * NP, LANES), F32),
        mesh=mesh,
        scratch_types=(
            pltpu.VMEM_SHARED((NP, LANES), F32),
            pltpu.VMEM((EBD,), jnp.int32),
            pltpu.VMEM((EBD, LANES), F32),
            pltpu.VMEM((ZR, LANES), F32),
        ),
    )


def _dotT(a, b):
    return lax.dot_general(a, b, (((1,), (1,)), ((), ())),
                           preferred_element_type=F32)


def _k1_body(n_chunks, *refs):
    agg_refs = refs[:n_chunks]
    d0_ref, d1_ref = refs[n_chunks:n_chunks + 2]
    h_refs = refs[n_chunks + 2: 2 * n_chunks + 2]
    wl_ref, wr_ref, bl_ref = refs[2 * n_chunks + 2: 2 * n_chunks + 5]
    z_ref, st_ref = refs[2 * n_chunks + 5:]

    inv = 1.0 / jnp.maximum(d0_ref[...] + d1_ref[...], 1.0)
    z = jnp.broadcast_to(bl_ref[...], (BN, 512)).astype(F32)
    for c in range(n_chunks):
        cs = slice(c * LANES, (c + 1) * LANES)
        z = z + _dotT(agg_refs[c][0] * inv, wl_ref[:, cs])
        z = z + _dotT(h_refs[c][0], wr_ref[:, cs])
    z_ref[...] = z

    s1 = jnp.sum(z, axis=0, keepdims=True)
    s2 = jnp.sum(z * z, axis=0, keepdims=True)
    st = jnp.concatenate([s1, s2], axis=0)

    @pl.when(pl.program_id(0) == 0)
    def _():
        st_ref[...] = jnp.zeros_like(st_ref)
    st_ref[...] += st


@functools.lru_cache(maxsize=None)
def _make_k1(n_chunks):
    grid = (GB,)
    in_specs = []
    for c in range(n_chunks):
        in_specs.append(pl.BlockSpec((1, BN, LANES),
                                     lambda i, c=c: (c, i, 0)))
    in_specs.append(pl.BlockSpec((BN, 1), lambda i: (i, 0)))
    in_specs.append(pl.BlockSpec((BN, 1), lambda i: (i, 0)))
    for c in range(n_chunks):
        in_specs.append(pl.BlockSpec((1, BN, LANES),
                                     lambda i, c=c: (c, i, 0)))
    din = n_chunks * LANES
    in_specs.append(pl.BlockSpec((512, din), lambda i: (0, 0)))
    in_specs.append(pl.BlockSpec((512, din), lambda i: (0, 0)))
    in_specs.append(pl.BlockSpec((1, 512), lambda i: (0, 0)))
    out_specs = [
        pl.BlockSpec((BN, 512), lambda i: (i, 0)),
        pl.BlockSpec((2, 512), lambda i: (0, 0)),
    ]
    return pl.pallas_call(
        functools.partial(_k1_body, n_chunks),
        grid=grid,
        in_specs=in_specs,
        out_specs=out_specs,
        out_shape=[
            jax.ShapeDtypeStruct((N, 512), F32),
            jax.ShapeDtypeStruct((2, 512), F32),
        ],
    )


def _k2_body(z_ref, st_ref, w_ref, b_ref, ms_ref, out_ref):
    st = st_ref[...]
    mean = st[0:1, :] * (1.0 / N)
    ex2 = st[1:2, :] * (1.0 / N)
    ms = ms_ref[...]
    var = ex2 - ms * mean * (2.0 * mean - ms * mean)
    a = w_ref[...] * lax.rsqrt(var + 1e-5)
    c2 = b_ref[...] - a * ms * mean
    h = jnp.maximum(z_ref[...] * a + c2, 0.0)
    out_ref[...] = h.reshape(1, BN, LANES)


def _make_k2():
    grid = (GB, 4)
    in_specs = [
        pl.BlockSpec((BN, LANES), lambda i, c: (i, c)),
        pl.BlockSpec((2, LANES), lambda i, c: (0, c)),
        pl.BlockSpec((1, LANES), lambda i, c: (0, c)),
        pl.BlockSpec((1, LANES), lambda i, c: (0, c)),
        pl.BlockSpec((1, LANES), lambda i, c: (0, c)),
    ]
    out_specs = pl.BlockSpec((1, BN, LANES), lambda i, c: (c, i, 0))
    return pl.pallas_call(
        _k2_body,
        grid=grid,
        in_specs=in_specs,
        out_specs=out_specs,
        out_shape=jax.ShapeDtypeStruct((4, NP, LANES), F32),
    )


def _k3_body(*refs):
    h_refs = refs[:4]
    w1_refs = refs[4:9]
    b1_refs = refs[9:14]
    w2_ref = refs[14]
    b2_ref = refs[15]
    out_ref = refs[16]

    acc = jnp.broadcast_to(b2_ref[...], (BN, LANES)).astype(F32)
    for hd in range(5):
        t = jnp.broadcast_to(b1_refs[hd][...], (BN, 512)).astype(F32)
        for c in range(4):
            cs = slice(c * LANES, (c + 1) * LANES)
            t = t + _dotT(h_refs[c][0], w1_refs[hd][:, cs])
        t = jnp.maximum(t, 0.0)
        acc = acc + _dotT(t, w2_ref[hd])
    out_ref[...] = acc


def _make_k3():
    grid = (GB,)
    in_specs = []
    for c in range(4):
        in_specs.append(pl.BlockSpec((1, BN, LANES),
                                     lambda i, c=c: (c, i, 0)))
    for _ in range(5):
        in_specs.append(pl.BlockSpec((512, 512), lambda i: (0, 0)))
    for _ in range(5):
        in_specs.append(pl.BlockSpec((1, 512), lambda i: (0, 0)))
    in_specs.append(pl.BlockSpec((5, LANES, 512), lambda i: (0, 0, 0)))
    in_specs.append(pl.BlockSpec((1, LANES), lambda i: (0, 0)))
    out_specs = pl.BlockSpec((BN, LANES), lambda i: (i, 0))
    return pl.pallas_call(
        _k3_body,
        grid=grid,
        in_specs=in_specs,
        out_specs=out_specs,
        out_shape=jax.ShapeDtypeStruct((N, LANES), F32),
    )


_OUT_DIMS = {'age': 21, 'sex': 2, 'ethnicity': 5, 'religion': 9, 'marital': 6}


def kernel(x, edge_index, params):
    src = edge_index[0].astype(jnp.int32)
    dst = edge_index[1].astype(jnp.int32)

    xt = x.reshape(N, 2, LANES).transpose(1, 0, 2)
    x2 = jnp.concatenate(
        [xt, jnp.zeros((2, NP - N, LANES), F32)], axis=1)
    src_off4 = jnp.concatenate([src + k * NP for k in range(4)])

    degp = _make_sc_deg()(dst)
    d0 = degp[:N, :1]
    d1 = degp[NP:NP + N, :1]
    agg2 = _make_sc_agg(2)(
        x2.reshape(2 * NP, LANES), src_off4[:2 * E], dst)
    agg2 = agg2.reshape(2, NP, LANES)

    z, st = _make_k1(2)(
        agg2, agg2, d0, d1, x2, x2,
        params['conv1']['Wl'], params['conv1']['Wr'],
        params['conv1']['bl'].reshape(1, 512))
    np1 = params['norm1']
    h_flat = _make_k2()(z, st, np1['weight'].reshape(1, 512),
                        np1['bias'].reshape(1, 512),
                        np1['mean_scale'].reshape(1, 512))

    for li in (2, 3, 4):
        agg4 = _make_sc_agg(4)(
            h_flat.reshape(4 * NP, LANES), src_off4, dst)
        agg4 = agg4.reshape(4, NP, LANES)
        cp = params['conv%d' % li]
        z, st = _make_k1(4)(
            agg4, agg4, agg4, agg4, d0, d1, h_flat, h_flat, h_flat, h_flat,
            cp['Wl'], cp['Wr'], cp['bl'].reshape(1, 512))
        npi = params['norm%d' % li]
        h_flat = _make_k2()(z, st, npi['weight'].reshape(1, 512),
                            npi['bias'].reshape(1, 512),
                            npi['mean_scale'].reshape(1, 512))

    w2pad = jnp.zeros((5, LANES, 512), F32)
    b2pad = jnp.zeros((1, LANES), F32)
    off = 0
    offs = []
    for hd, (name, od) in enumerate(_OUT_DIMS.items()):
        mp = params['mlp_%s' % name]
        w2pad = w2pad.at[hd, off:off + od, :].set(mp['W2'])
        b2pad = b2pad.at[0, off:off + od].set(mp['b2'])
        offs.append(off)
        off += od

    w1s = [params['mlp_%s' % n]['W1'] for n in _OUT_DIMS]
    b1s = [params['mlp_%s' % n]['b1'].reshape(1, 512) for n in _OUT_DIMS]
    ocat = _make_k3()(h_flat, h_flat, h_flat, h_flat, *w1s, *b1s,
                      w2pad, b2pad)

    outs = []
    for (name, od), o0 in zip(_OUT_DIMS.items(), offs):
        outs.append(ocat[:, o0:o0 + od])
    return tuple(outs)


# speedup vs baseline: 3.3401x; 1.2087x over previous
import functools

import jax
import jax.numpy as jnp
from jax import lax
from jax.experimental import pallas as pl
from jax.experimental.pallas import tpu as pltpu
from jax.experimental.pallas import tpu_sc as plsc

N = 10000
E = 160000
NP = 10240
LANES = 128
NC = 2
NS = 16
L = 16
EB = 128
ET = E // NS
EP = 10240
NB = EP // EB
HB = NB // 2
ETD = E // (NC * NS)
EPD = 5120
NBD = EPD // EB
RPT = NP // NS
ZR = 128
DW = 128
BN = 1000
GB = N // BN
F32 = jnp.float32


def _sc_body_factory(n_chunks):
    cpc = n_chunks // NC

    def body(h_hbm, src_hbm, dst_hbm, out_hbm,
             acc, sidx, didx, rows0, rows1, sem0, sem1):
        cid = lax.axis_index("c")
        sid = lax.axis_index("s")
        row0 = pl.multiple_of(sid * RPT, ZR)
        rows = (rows0, rows1)
        sems = (sem0, sem1)
        zeros16 = jnp.zeros((L,), F32)

        pltpu.sync_copy(dst_hbm.at[sid], didx)

        for j in range(cpc):
            chunk = cid + NC * j

            def _zrow(r, _):
                def _zlane(q, _):
                    rows0[r, pl.ds(q * L, L)] = zeros16
                    return 0
                return lax.fori_loop(0, LANES // L, _zlane, 0)
            lax.fori_loop(0, ZR, _zrow, 0)
            for k in range(RPT // ZR):
                pltpu.sync_copy(rows0, acc.at[pl.ds(row0 + k * ZR, ZR)])
            plsc.subcore_barrier()

            for hh in range(2):
                pltpu.sync_copy(
                    src_hbm.at[chunk].at[sid].at[pl.ds(hh * HB, HB)], sidx)

                def fire(i, b):
                    pltpu.async_copy(h_hbm.at[sidx.at[i]], rows[b], sems[b])

                def drain_scatter(i, b):
                    pltpu.make_async_copy(
                        h_hbm.at[sidx.at[i]], rows[b], sems[b]).wait()
                    pltpu.sync_copy(rows[b], acc.at[didx.at[i + hh * HB]],
                                    add=True)

                fire(0, 0)
                fire(1, 1)

                def _pair(i2, _):
                    i = i2 * 2
                    for b in range(2):
                        drain_scatter(i + b, b)
                        fire(i + b + 2, b)
                    return 0
                lax.fori_loop(0, (HB - 2) // 2, _pair, 0)
                for b in range(2):
                    drain_scatter(HB - 2 + b, b)
            plsc.subcore_barrier()

            base = pl.multiple_of(chunk * NP + row0, ZR)
            for k in range(RPT // ZR):
                pltpu.sync_copy(
                    acc.at[pl.ds(row0 + k * ZR, ZR)],
                    out_hbm.at[pl.ds(base + k * ZR, ZR)])

    return body


@functools.lru_cache(maxsize=None)
def _make_sc_agg(n_chunks):
    mesh = plsc.VectorSubcoreMesh(core_axis_name="c", subcore_axis_name="s")
    return pl.kernel(
        _sc_body_factory(n_chunks),
        out_type=jax.ShapeDtypeStruct((n_chunks * NP, LANES), F32),
        mesh=mesh,
        scratch_types=(
            pltpu.VMEM_SHARED((NP, LANES), F32),
            pltpu.VMEM((HB, EB), jnp.int32),
            pltpu.VMEM((NB, EB), jnp.int32),
            pltpu.VMEM((EB, LANES), F32),
            pltpu.VMEM((EB, LANES), F32),
            pltpu.SemaphoreType.DMA,
            pltpu.SemaphoreType.DMA,
        ),
    )


def _sc_deg_body(dst_hbm, deg_hbm, acc, didx, ones, zb):
    cid = lax.axis_index("c")
    sid = lax.axis_index("s")
    row0 = pl.multiple_of(sid * RPT, ZR)
    zeros16 = jnp.zeros((L,), F32)
    ones16 = jnp.ones((L,), F32)

    def _fill(ref, nrows, val):
        def _row(r, _):
            def _lane(q, _):
                ref[r, pl.ds(q * L, L)] = val
                return 0
            return lax.fori_loop(0, DW // L, _lane, 0)
        lax.fori_loop(0, nrows, _row, 0)

    _fill(ones, EB, ones16)
    _fill(zb, ZR, zeros16)
    for k in range(RPT // ZR):
        pltpu.sync_copy(zb, acc.at[pl.ds(row0 + k * ZR, ZR)])
    plsc.subcore_barrier()

    pltpu.sync_copy(dst_hbm.at[cid * NS + sid], didx)

    def _edge(i, _):
        pltpu.sync_copy(ones, acc.at[didx.at[i]], add=True)
        return 0
    lax.fori_loop(0, NBD, _edge, 0)
    plsc.subcore_barrier()

    base = pl.multiple_of(cid * NP + row0, ZR)
    for k in range(RPT // ZR):
        pltpu.sync_copy(acc.at[pl.ds(row0 + k * ZR, ZR)],
                        deg_hbm.at[pl.ds(base + k * ZR, ZR)])


def _make_sc_deg():
    mesh = plsc.VectorSubcoreMesh(core_axis_name="c", subcore_axis_name="s")
    return pl.kernel(
        _sc_deg_body,
        out_type=jax.ShapeDtypeStruct((NC * NP, DW), F32),
        mesh=mesh,
        scratch_types=(
            pltpu.VMEM_SHARED((NP, DW), F32),
            pltpu.VMEM((NBD, EB), jnp.int32),
            pltpu.VMEM((EB, DW), F32),
            pltpu.VMEM((ZR, DW), F32),
        ),
    )


def _dotT(a, b):
    return lax.dot_general(a, b, (((1,), (1,)), ((), ())),
                           preferred_element_type=F32)


def _k1_body(n_chunks, *refs):
    agg_refs = refs[:n_chunks]
    d0_ref, d1_ref = refs[n_chunks:n_chunks + 2]
    h_refs = refs[n_chunks + 2: 2 * n_chunks + 2]
    wl_ref, wr_ref, bl_ref = refs[2 * n_chunks + 2: 2 * n_chunks + 5]
    z_ref, st_ref = refs[2 * n_chunks + 5:]

    inv = 1.0 / jnp.maximum(d0_ref[...] + d1_ref[...], 1.0)
    z = jnp.broadcast_to(bl_ref[...], (BN, 512)).astype(F32)
    for c in range(n_chunks):
        cs = slice(c * LANES, (c + 1) * LANES)
        z = z + _dotT(agg_refs[c][0] * inv, wl_ref[:, cs])
        z = z + _dotT(h_refs[c][0], wr_ref[:, cs])
    z_ref[...] = z

    s1 = jnp.sum(z, axis=0, keepdims=True)
    s2 = jnp.sum(z * z, axis=0, keepdims=True)
    st = jnp.concatenate([s1, s2], axis=0)

    @pl.when(pl.program_id(0) == 0)
    def _():
        st_ref[...] = jnp.zeros_like(st_ref)
    st_ref[...] += st


@functools.lru_cache(maxsize=None)
def _make_k1(n_chunks):
    grid = (GB,)
    in_specs = []
    for c in range(n_chunks):
        in_specs.append(pl.BlockSpec((1, BN, LANES),
                                     lambda i, c=c: (c, i, 0)))
    in_specs.append(pl.BlockSpec((BN, 1), lambda i: (i, 0)))
    in_specs.append(pl.BlockSpec((BN, 1), lambda i: (i, 0)))
    for c in range(n_chunks):
        in_specs.append(pl.BlockSpec((1, BN, LANES),
                                     lambda i, c=c: (c, i, 0)))
    din = n_chunks * LANES
    in_specs.append(pl.BlockSpec((512, din), lambda i: (0, 0)))
    in_specs.append(pl.BlockSpec((512, din), lambda i: (0, 0)))
    in_specs.append(pl.BlockSpec((1, 512), lambda i: (0, 0)))
    out_specs = [
        pl.BlockSpec((BN, 512), lambda i: (i, 0)),
        pl.BlockSpec((2, 512), lambda i: (0, 0)),
    ]
    return pl.pallas_call(
        functools.partial(_k1_body, n_chunks),
        grid=grid,
        in_specs=in_specs,
        out_specs=out_specs,
        out_shape=[
            jax.ShapeDtypeStruct((N, 512), F32),
            jax.ShapeDtypeStruct((2, 512), F32),
        ],
    )


def _k2_body(z_ref, st_ref, w_ref, b_ref, ms_ref, out_ref):
    st = st_ref[...]
    mean = st[0:1, :] * (1.0 / N)
    ex2 = st[1:2, :] * (1.0 / N)
    ms = ms_ref[...]
    var = ex2 - ms * mean * (2.0 * mean - ms * mean)
    a = w_ref[...] * lax.rsqrt(var + 1e-5)
    c2 = b_ref[...] - a * ms * mean
    h = jnp.maximum(z_ref[...] * a + c2, 0.0)
    out_ref[...] = h.reshape(1, BN, LANES)


def _make_k2():
    grid = (GB, 4)
    in_specs = [
        pl.BlockSpec((BN, LANES), lambda i, c: (i, c)),
        pl.BlockSpec((2, LANES), lambda i, c: (0, c)),
        pl.BlockSpec((1, LANES), lambda i, c: (0, c)),
        pl.BlockSpec((1, LANES), lambda i, c: (0, c)),
        pl.BlockSpec((1, LANES), lambda i, c: (0, c)),
    ]
    out_specs = pl.BlockSpec((1, BN, LANES), lambda i, c: (c, i, 0))
    return pl.pallas_call(
        _k2_body,
        grid=grid,
        in_specs=in_specs,
        out_specs=out_specs,
        out_shape=jax.ShapeDtypeStruct((4, NP, LANES), F32),
    )


def _k3_body(*refs):
    h_refs = refs[:4]
    w1_refs = refs[4:9]
    b1_refs = refs[9:14]
    w2_ref = refs[14]
    b2_ref = refs[15]
    out_ref = refs[16]

    acc = jnp.broadcast_to(b2_ref[...], (BN, LANES)).astype(F32)
    for hd in range(5):
        t = jnp.broadcast_to(b1_refs[hd][...], (BN, 512)).astype(F32)
        for c in range(4):
            cs = slice(c * LANES, (c + 1) * LANES)
            t = t + _dotT(h_refs[c][0], w1_refs[hd][:, cs])
        t = jnp.maximum(t, 0.0)
        acc = acc + _dotT(t, w2_ref[hd])
    out_ref[...] = acc


def _make_k3():
    grid = (GB,)
    in_specs = []
    for c in range(4):
        in_specs.append(pl.BlockSpec((1, BN, LANES),
                                     lambda i, c=c: (c, i, 0)))
    for _ in range(5):
        in_specs.append(pl.BlockSpec((512, 512), lambda i: (0, 0)))
    for _ in range(5):
        in_specs.append(pl.BlockSpec((1, 512), lambda i: (0, 0)))
    in_specs.append(pl.BlockSpec((5, LANES, 512), lambda i: (0, 0, 0)))
    in_specs.append(pl.BlockSpec((1, LANES), lambda i: (0, 0)))
    out_specs = pl.BlockSpec((BN, LANES), lambda i: (i, 0))
    return pl.pallas_call(
        _k3_body,
        grid=grid,
        in_specs=in_specs,
        out_specs=out_specs,
        out_shape=jax.ShapeDtypeStruct((N, LANES), F32),
    )


_OUT_DIMS = {'age': 21, 'sex': 2, 'ethnicity': 5, 'religion': 9, 'marital': 6}


def kernel(x, edge_index, params):
    src = edge_index[0].astype(jnp.int32)
    dst = edge_index[1].astype(jnp.int32)

    xt = x.reshape(N, 2, LANES).transpose(1, 0, 2)
    x2 = jnp.concatenate(
        [xt, jnp.zeros((2, NP - N, LANES), F32)], axis=1)
    srcp = jnp.concatenate(
        [src.reshape(NS, ET),
         jnp.full((NS, EP - ET), N, jnp.int32)], axis=1)
    src_off4 = jnp.stack(
        [srcp + k * NP for k in range(4)]).reshape(4, NS, NB, EB)
    dst3 = jnp.concatenate(
        [dst.reshape(NS, ET),
         jnp.full((NS, EP - ET), NP - 1, jnp.int32)], axis=1
    ).reshape(NS, NB, EB)
    dstd = jnp.concatenate(
        [dst.reshape(NC * NS, ETD),
         jnp.full((NC * NS, EPD - ETD), NP - 1, jnp.int32)], axis=1
    ).reshape(NC * NS, NBD, EB)

    degp = _make_sc_deg()(dstd)
    d0 = degp[:N, :1]
    d1 = degp[NP:NP + N, :1]
    agg2 = _make_sc_agg(2)(
        x2.reshape(2 * NP, LANES), src_off4[:2], dst3)
    agg2 = agg2.reshape(2, NP, LANES)

    z, st = _make_k1(2)(
        agg2, agg2, d0, d1, x2, x2,
        params['conv1']['Wl'], params['conv1']['Wr'],
        params['conv1']['bl'].reshape(1, 512))
    np1 = params['norm1']
    h_flat = _make_k2()(z, st, np1['weight'].reshape(1, 512),
                        np1['bias'].reshape(1, 512),
                        np1['mean_scale'].reshape(1, 512))

    for li in (2, 3, 4):
        agg4 = _make_sc_agg(4)(
            h_flat.reshape(4 * NP, LANES), src_off4, dst3)
        agg4 = agg4.reshape(4, NP, LANES)
        cp = params['conv%d' % li]
        z, st = _make_k1(4)(
            agg4, agg4, agg4, agg4, d0, d1, h_flat, h_flat, h_flat, h_flat,
            cp['Wl'], cp['Wr'], cp['bl'].reshape(1, 512))
        npi = params['norm%d' % li]
        h_flat = _make_k2()(z, st, npi['weight'].reshape(1, 512),
                            npi['bias'].reshape(1, 512),
                            npi['mean_scale'].reshape(1, 512))

    w2pad = jnp.zeros((5, LANES, 512), F32)
    b2pad = jnp.zeros((1, LANES), F32)
    off = 0
    offs = []
    for hd, (name, od) in enumerate(_OUT_DIMS.items()):
        mp = params['mlp_%s' % name]
        w2pad = w2pad.at[hd, off:off + od, :].set(mp['W2'])
        b2pad = b2pad.at[0, off:off + od].set(mp['b2'])
        offs.append(off)
        off += od

    w1s = [params['mlp_%s' % n]['W1'] for n in _OUT_DIMS]
    b1s = [params['mlp_%s' % n]['b1'].reshape(1, 512) for n in _OUT_DIMS]
    ocat = _make_k3()(h_flat, h_flat, h_flat, h_flat, *w1s, *b1s,
                      w2pad, b2pad)

    outs = []
    for (name, od), o0 in zip(_OUT_DIMS.items(), offs):
        outs.append(ocat[:, o0:o0 + od])
    return tuple(outs)
